# Initial kernel scaffold; baseline (speedup 1.0000x reference)
import jax
import jax.numpy as jnp
from jax.experimental import pallas as pl


def kernel(s, v, edge_index, edge_dist, edge_vec, batch, params):
    raise NotImplementedError("write your pallas kernel here")


# speedup vs baseline: 16.3075x; 16.3075x over previous
import functools

import jax
import jax.numpy as jnp
from jax import lax
from jax.experimental import pallas as pl
from jax.experimental.pallas import tpu as pltpu
from jax.experimental.pallas import tpu_sc as plsc

DS, DV, RB = 128, 32, 32
DV3 = 3 * DV
DSV = DS + 2 * DV
CUT = 5.0
EPS = 1e-6
NC, NSUB = 2, 16
C = 80


def _phi_body(s_ref, w1_ref, b1_ref, w2_ref, b2_ref, ps_ref, pv_ref):
    h = s_ref[...] @ w1_ref[...] + b1_ref[...]
    h = h * jax.nn.sigmoid(h)
    phi = h @ w2_ref[...] + b2_ref[...]
    ps_ref[...] = phi[:, :DS]
    pv_ref[:, :2 * DV] = phi[:, DS:]
    pv_ref[:, 2 * DV:] = jnp.zeros_like(phi[:, DS:])


def _phi_call(s, w1, b1, w2, b2):
    n = s.shape[0]
    bn = 1000
    return pl.pallas_call(
        _phi_body,
        grid=(n // bn,),
        in_specs=[
            pl.BlockSpec((bn, DS), lambda i: (i, 0)),
            pl.BlockSpec((DS, DS), lambda i: (0, 0)),
            pl.BlockSpec((1, DS), lambda i: (0, 0)),
            pl.BlockSpec((DS, DSV), lambda i: (0, 0)),
            pl.BlockSpec((1, DSV), lambda i: (0, 0)),
        ],
        out_specs=[
            pl.BlockSpec((bn, DS), lambda i: (i, 0)),
            pl.BlockSpec((bn, DS), lambda i: (i, 0)),
        ],
        out_shape=[
            jax.ShapeDtypeStruct((n, DS), jnp.float32),
            jax.ShapeDtypeStruct((n, DS), jnp.float32),
        ],
    )(s, w1, b1, w2, b2)


def _filt_body(d_ref, ev_ref, wr_ref, br_ref, fs_ref, fv_ref):
    d = d_ref[...]
    dc = jnp.maximum(d, 1e-3)
    n = (lax.broadcasted_iota(jnp.int32, (1, RB), 1) + 1).astype(jnp.float32)
    feats = jnp.sin(n * (jnp.pi / CUT) * dc) / dc
    cutf = jnp.where(d < CUT, 0.5 * (jnp.cos(jnp.pi / CUT * d) + 1.0), 0.0)
    f = (feats @ wr_ref[...] + br_ref[...]) * cutf
    fs_ref[...] = f[:, :DS]
    fv_ref[:, :2 * DV] = f[:, DS:]
    ev = ev_ref[...]
    nrm = jnp.sqrt(jnp.sum(ev * ev, axis=1, keepdims=True))
    dirv = ev / (nrm + EPS)
    be = ev.shape[0]
    fv_ref[:, 2 * DV:2 * DV + 3] = dirv
    fv_ref[:, 2 * DV + 3:] = jnp.zeros((be, 2 * DV - 3), jnp.float32)


def _filt_call(d2, ev, wr, br):
    e = d2.shape[0]
    be = 2000
    return pl.pallas_call(
        _filt_body,
        grid=(e // be,),
        in_specs=[
            pl.BlockSpec((be, 1), lambda i: (i, 0)),
            pl.BlockSpec((be, 3), lambda i: (i, 0)),
            pl.BlockSpec((RB, DSV), lambda i: (0, 0)),
            pl.BlockSpec((1, DSV), lambda i: (0, 0)),
        ],
        out_specs=[
            pl.BlockSpec((be, DS), lambda i: (i, 0)),
            pl.BlockSpec((be, DS), lambda i: (i, 0)),
        ],
        out_shape=[
            jax.ShapeDtypeStruct((e, DS), jnp.float32),
            jax.ShapeDtypeStruct((e, DS), jnp.float32),
        ],
    )(d2, ev, wr, br)


def _make_sc(n, e, has_v):
    nz = 8
    npad = ((n + NSUB * nz - 1) // (NSUB * nz)) * (NSUB * nz)
    nper = npad // NSUB
    epw = e // NSUB
    nch = epw // C
    assert nper % nz == 0 and epw % C == 0

    mesh = plsc.VectorSubcoreMesh(core_axis_name="c", subcore_axis_name="s",
                                  num_cores=NC, num_subcores=NSUB)

    @functools.partial(
        pl.kernel,
        out_type=[
            jax.ShapeDtypeStruct((npad, DS), jnp.float32),
            jax.ShapeDtypeStruct((npad, DS), jnp.float32),
        ],
        mesh=mesh,
        scratch_types=[
            pltpu.VMEM_SHARED((npad, DS), jnp.float32),
            pltpu.VMEM((C,), jnp.int32),
            pltpu.VMEM((C,), jnp.int32),
            pltpu.VMEM((C, DS), jnp.float32),
            pltpu.VMEM((C, DS), jnp.float32),
            pltpu.VMEM((C, DS), jnp.float32),
            pltpu.VMEM((C, DS), jnp.float32),
            pltpu.VMEM((nz, DS), jnp.float32),
            pltpu.SemaphoreType.DMA,
            pltpu.SemaphoreType.DMA,
            pltpu.SemaphoreType.DMA,
            pltpu.SemaphoreType.DMA,
        ],
    )
    def sc_kernel(phis, phiv, vflat, fs, fv, src, dst,
                  agg0, agg1,
                  acc, srcb, dstb, pb, fb, mb, vg, zbuf,
                  sem_a, sem_b, sem_c, sem_d):
        cid = lax.axis_index("c")
        sid = lax.axis_index("s")
        base = sid * nper
        z16 = jnp.zeros((16,), jnp.float32)

        def zrow(r, carry):
            for k in range(DS // 16):
                zbuf[r, pl.ds(k * 16, 16)] = z16
            return carry
        lax.fori_loop(0, nz, zrow, 0)
        for j in range(nper // nz):
            pltpu.sync_copy(zbuf, acc.at[pl.ds(base + j * nz, nz)])
        plsc.subcore_barrier()

        @pl.when(cid == 1)
        def _():
            ione = jnp.where(lax.iota(jnp.int32, 16) == 0,
                             jnp.float32(1.0), jnp.float32(0.0))

            def prep(r, carry):
                mb[r, pl.ds(DV3, 16)] = ione
                mb[r, pl.ds(DV3 + 16, 16)] = z16
                return carry
            lax.fori_loop(0, C, prep, 0)

        ebase = sid * epw

        def chunk(i, carry):
            e0 = ebase + i * C
            pltpu.sync_copy(src.at[pl.ds(e0, C)], srcb)
            pltpu.sync_copy(dst.at[pl.ds(e0, C)], dstb)

            @pl.when(cid == 0)
            def _():
                cp1 = pltpu.async_copy(phis.at[srcb], pb, sem_a)
                cp2 = pltpu.async_copy(fs.at[pl.ds(e0, C)], fb, sem_b)
                cp1.wait()
                cp2.wait()

                def row(r, rc):
                    for k in range(DS // 16):
                        sl = pl.ds(k * 16, 16)
                        mb[r, sl] = pb[r, sl] * fb[r, sl]
                    return rc
                lax.fori_loop(0, C, row, 0)
                pltpu.sync_copy(mb, acc.at[dstb], add=True)

            @pl.when(cid == 1)
            def _():
                cp1 = pltpu.async_copy(phiv.at[srcb], pb, sem_a)
                cp2 = pltpu.async_copy(fv.at[pl.ds(e0, C)], fb, sem_b)
                if has_v:
                    cp4 = pltpu.async_copy(vflat.at[srcb], vg, sem_d)
                cp1.wait()
                cp2.wait()
                if has_v:
                    cp4.wait()

                def row(r, rc):
                    dvec = fb[r, pl.ds(2 * DV, 16)]
                    dval = [dvec[d] for d in range(3)]
                    for k in range(DV // 16):
                        sl1 = pl.ds(k * 16, 16)
                        sl2 = pl.ds(DV + k * 16, 16)
                        mv1 = pb[r, sl1] * fb[r, sl1]
                        mv2 = pb[r, sl2] * fb[r, sl2]
                        for d in range(3):
                            out = mv2 * dval[d]
                            if has_v:
                                out = out + mv1 * vg[r, pl.ds(d * DV + k * 16, 16)]
                            mb[r, pl.ds(d * DV + k * 16, 16)] = out
                    return rc
                lax.fori_loop(0, C, row, 0)
                pltpu.sync_copy(mb, acc.at[dstb], add=True)

            return carry
        lax.fori_loop(0, nch, chunk, 0)
        plsc.subcore_barrier()

        for j in range(nper // nz):
            sl = pl.ds(base + j * nz, nz)
            pltpu.sync_copy(acc.at[sl], zbuf)

            @pl.when(cid == 0)
            def _():
                pltpu.sync_copy(zbuf, agg0.at[sl])

            @pl.when(cid == 1)
            def _():
                pltpu.sync_copy(zbuf, agg1.at[sl])

    return sc_kernel


def _upd_body(s_ref, vf_ref, a0_ref, a1_ref, wu_ref, wv_ref, wu1s_ref,
              wu1v_ref, b1_ref, wu2_ref, b2_ref, wsp_ref, so_ref, vo_ref):
    a1 = a1_ref[...]
    denom = jnp.maximum(a1[:, DV3:DV3 + 1], 1.0)
    s1 = s_ref[...] + a0_ref[...] / denom
    v1 = vf_ref[:, :DV3] + a1[:, :DV3] / denom
    u = v1 @ wu_ref[...]
    vv = v1 @ wv_ref[...]
    vvsq = vv * vv
    vn = jnp.sqrt(vvsq[:, :DV] + vvsq[:, DV:2 * DV] + vvsq[:, 2 * DV:] + EPS)
    pre = s1 @ wu1s_ref[...] + vn @ wu1v_ref[...] + b1_ref[...]
    aa = pre * jax.nn.sigmoid(pre)
    aa = aa @ wu2_ref[...] + b2_ref[...]
    a_ss = aa[:, :DS]
    a_sv = aa[:, DS:DS + DV]
    a_vv = aa[:, DS + DV:]
    uvv = u * vv
    sp = uvv[:, :DV] + uvv[:, DV:2 * DV] + uvv[:, 2 * DV:]
    so_ref[...] = s1 + a_ss + (a_sv * sp) @ wsp_ref[...]
    dv = jnp.concatenate(
        [a_vv * u[:, :DV], a_vv * u[:, DV:2 * DV], a_vv * u[:, 2 * DV:]],
        axis=1)
    vo_ref[:, :DV3] = v1 + dv
    vo_ref[:, DV3:] = jnp.zeros_like(a_sv)


def _upd_call(s, vf, a0, a1, wu, wv, wu1s, wu1v, b1, wu2, b2, wsp):
    n = s.shape[0]
    bn = 1000
    full = lambda shape: pl.BlockSpec(shape, lambda i: (0, 0))
    row = lambda w: pl.BlockSpec((bn, w), lambda i: (i, 0))
    return pl.pallas_call(
        _upd_body,
        grid=(n // bn,),
        in_specs=[
            row(DS), row(DS), row(DS), row(DS),
            full((DV3, DV3)), full((DV3, DV3)),
            full((DS, DS)), full((DV, DS)), full((1, DS)),
            full((DS, DSV)), full((1, DSV)), full((DV, DS)),
        ],
        out_specs=[row(DS), row(DS)],
        out_shape=[
            jax.ShapeDtypeStruct((n, DS), jnp.float32),
            jax.ShapeDtypeStruct((n, DS), jnp.float32),
        ],
    )(s, vf, a0, a1, wu, wv, wu1s, wu1v, b1, wu2, b2, wsp)


def _bd3(w):
    z = jnp.zeros((DV, DV), w.dtype)
    return jnp.block([[w, z, z], [z, w, z], [z, z, w]])


def kernel(s, v, edge_index, edge_dist, edge_vec, batch, params):
    n = s.shape[0]
    e = edge_dist.shape[0]
    src = edge_index[0]
    dst = edge_index[1]
    vflat = jnp.pad(v.reshape(n, DV3), ((0, 0), (0, DS - DV3)))
    d2 = edge_dist.reshape(e, 1)

    sc_first = _make_sc(n, e, has_v=False)
    sc_rest = _make_sc(n, e, has_v=True)

    for i, p in enumerate(params):
        phis, phiv = _phi_call(s, p['W_phi1'], p['b_phi1'].reshape(1, DS),
                               p['W_phi2'], p['b_phi2'].reshape(1, DSV))
        fs, fv = _filt_call(d2, edge_vec, p['W_rbf'], p['b_rbf'].reshape(1, DSV))
        sc = sc_first if i == 0 else sc_rest
        agg0, agg1 = sc(phis, phiv, vflat, fs, fv, src, dst)
        s, vflat = _upd_call(
            s, vflat, agg0[:n], agg1[:n],
            _bd3(p['W_U']), _bd3(p['W_V']),
            p['W_u1'][:DS], p['W_u1'][DS:], p['b_u1'].reshape(1, DS),
            p['W_u2'], p['b_u2'].reshape(1, DSV), p['W_sp'],
        )

    return s, vflat[:, :DV3].reshape(n, 3, DV)
